# baseline (speedup 1.0000x reference)
import functools

import jax
import jax.numpy as jnp
from jax import lax
from jax.experimental import pallas as pl
from jax.experimental.pallas import tpu as pltpu
from jax.experimental.pallas import tpu_sc as plsc

_NTOKEN = 1000
_NC = 2
_NS = 16
_NW = _NC * _NS
_LANES = 16

_SC_SLABS = _NW
_CHUNK_C = 40
_NCHUNK = _NTOKEN // _CHUNK_C


def _tc_body(x_ref, out_ref):
    xb = x_ref[...]
    tok = lax.broadcasted_iota(jnp.int32, out_ref.shape, 1)
    out_ref[...] = (tok == xb).astype(jnp.float32)


def _tc_onehot(x_t, L):
    B = x_t.shape[2]
    return pl.pallas_call(
        _tc_body,
        grid=(L,),
        in_specs=[pl.BlockSpec((1, 1, B), lambda i: (i, 0, 0))],
        out_specs=pl.BlockSpec((1, _NTOKEN, B), lambda i: (i, 0, 0)),
        out_shape=jax.ShapeDtypeStruct((L, _NTOKEN, B), jnp.float32),
    )(x_t)


def _sc_poke(buf, xbuf, k, B, iota, val):
    c0 = k * _CHUNK_C
    for v in range(B // _LANES):
        b = v * _LANES + iota
        cv = xbuf[pl.ds(v * _LANES, _LANES)]
        m = (cv >= c0) & (cv < c0 + _CHUNK_C)
        plsc.store_scatter(buf, [cv - c0, b], val, mask=m)


def _onehot_sc(x_hbm, out_hbm, xbuf, buf0, buf1, sem0, sem1, *, B, l0):
    wid = lax.axis_index("s") * _NC + lax.axis_index("c")
    pltpu.sync_copy(x_hbm.at[pl.ds((l0 + wid) * B, B)], xbuf)

    iota = lax.iota(jnp.int32, _LANES)
    ones = jnp.full((_LANES,), 1.0, jnp.float32)
    zeros = jnp.zeros((_LANES,), jnp.float32)

    def _memset(r, c):
        for o in range(0, B, _LANES):
            buf0[r, pl.ds(o, _LANES)] = zeros
            buf1[r, pl.ds(o, _LANES)] = zeros
        return c

    lax.fori_loop(0, _CHUNK_C, _memset, 0)

    bufs = (buf0, buf1)
    sems = (sem0, sem1)

    def _dst(k):
        return out_hbm.at[wid, pl.ds(k * _CHUNK_C, _CHUNK_C), :]

    def _unit(k, r, primed):
        if primed:
            pltpu.make_async_copy(bufs[r], _dst(k - 2), sems[r]).wait()
            _sc_poke(bufs[r], xbuf, k - 2, B, iota, zeros)
        _sc_poke(bufs[r], xbuf, k, B, iota, ones)
        pltpu.async_copy(bufs[r], _dst(k), sems[r])

    _unit(0, 0, False)
    _unit(1, 1, False)

    def _step(g, c):
        _unit(2 * g, 0, True)
        _unit(2 * g + 1, 1, True)
        return c

    lax.fori_loop(1, _NCHUNK // 2, _step, 0)
    _unit(_NCHUNK - 1, 0, True)

    pltpu.make_async_copy(bufs[0], _dst(_NCHUNK - 1), sems[0]).wait()
    pltpu.make_async_copy(bufs[1], _dst(_NCHUNK - 2), sems[1]).wait()


def _sc_onehot(xf, B, l0):
    body = functools.partial(_onehot_sc, B=B, l0=l0)
    body.__name__ = "_onehot_sc"

    return pl.kernel(
        body,
        mesh=plsc.VectorSubcoreMesh(core_axis_name="c", subcore_axis_name="s"),
        compiler_params=pltpu.CompilerParams(
            needs_layout_passes=False, skip_device_barrier=True
        ),
        out_type=jax.ShapeDtypeStruct((_SC_SLABS, _NTOKEN, B), jnp.float32),
        scratch_types=[
            pltpu.VMEM((B,), jnp.int32),
            pltpu.VMEM((_CHUNK_C, B), jnp.float32),
            pltpu.VMEM((_CHUNK_C, B), jnp.float32),
            pltpu.SemaphoreType.DMA,
            pltpu.SemaphoreType.DMA,
        ],
    )(xf)


def kernel(x):
    B, L = x.shape
    x_t = x.T
    a = _tc_onehot(x_t.reshape(L, 1, B), L)
    b = _sc_onehot(x_t.reshape(L * B), B, L - _SC_SLABS)
    return a.transpose(2, 0, 1), b


# speedup vs baseline: 1.8535x; 1.8535x over previous
import jax
import jax.numpy as jnp
from jax import lax
from jax.experimental import pallas as pl

_NTOKEN = 1000


def _tc_body(x_ref, out_ref):
    xb = x_ref[...]
    tok = lax.broadcasted_iota(jnp.int32, out_ref.shape, 1)
    out_ref[...] = (tok == xb).astype(jnp.float32)


def kernel(x):
    B, L = x.shape
    x_t = x.T.reshape(L, 1, B)
    out_lcb = pl.pallas_call(
        _tc_body,
        grid=(L // 2,),
        in_specs=[pl.BlockSpec((2, 1, B), lambda i: (i, 0, 0))],
        out_specs=pl.BlockSpec((2, _NTOKEN, B), lambda i: (i, 0, 0)),
        out_shape=jax.ShapeDtypeStruct((L, _NTOKEN, B), jnp.float32),
    )(x_t)
    return out_lcb.transpose(2, 0, 1)


# speedup vs baseline: 1.8886x; 1.0189x over previous
import jax
import jax.numpy as jnp
from jax import lax
from jax.experimental import pallas as pl

_NTOKEN = 1000


def _tc_body(x_ref, out_ref):
    xb = x_ref[...]
    tok = lax.broadcasted_iota(jnp.int32, out_ref.shape, 1)
    out_ref[...] = (tok == xb).astype(jnp.float32)


def kernel(x):
    B, L = x.shape
    x_t = x.T.reshape(L, 1, B)
    out_lcb = pl.pallas_call(
        _tc_body,
        grid=(L,),
        in_specs=[pl.BlockSpec((1, 1, B), lambda i: (i, 0, 0))],
        out_specs=pl.BlockSpec((1, _NTOKEN, B), lambda i: (i, 0, 0)),
        out_shape=jax.ShapeDtypeStruct((L, _NTOKEN, B), jnp.float32),
    )(x_t)
    return out_lcb.transpose(2, 0, 1)
